# baseline (speedup 1.0000x reference)
import functools

import jax
import jax.numpy as jnp
from jax import lax
from jax.experimental import pallas as pl
from jax.experimental.pallas import tpu as pltpu
from jax.experimental.pallas import tpu_sc as plsc

N = 10000
NP = 10240
D = 128
NC = 2
NS = 16
CHUNK = 128
CH = 49
EPAD = NS * CH * CHUNK
ROWS_PT = NP // NS

_f32 = jnp.float32


def _mesh():
    return plsc.VectorSubcoreMesh(
        core_axis_name="c", subcore_axis_name="s", num_cores=NC, num_subcores=NS
    )


def _sc_aggregate(src3, dst3, feat, with_deg):
    R = src3.shape[0]
    out_type = [jax.ShapeDtypeStruct((R, NP, D), _f32)]
    if with_deg:
        out_type.append(jax.ShapeDtypeStruct((R, 1, NP), _f32))

    scratch = [
        pltpu.VMEM((CH, CHUNK), jnp.int32),
        pltpu.VMEM((CH, CHUNK), jnp.int32),
        pltpu.VMEM((2, CHUNK, D), _f32),
        pltpu.VMEM((ROWS_PT,), _f32),
        pltpu.VMEM((CHUNK,), _f32),
        pltpu.VMEM_SHARED((NP, D), _f32),
        pltpu.VMEM_SHARED((NP,), _f32),
        pltpu.SemaphoreType.DMA,
        pltpu.SemaphoreType.DMA,
        pltpu.SemaphoreType.DMA,
        pltpu.SemaphoreType.DMA,
        pltpu.SemaphoreType.DMA,
    ]

    @functools.partial(
        pl.kernel, out_type=tuple(out_type), mesh=_mesh(), scratch_types=scratch
    )
    def body(src_h, dst_h, feat_h, *outs_and_scratch):
        if with_deg:
            out_agg, out_deg = outs_and_scratch[:2]
            rest = outs_and_scratch[2:]
        else:
            out_agg = outs_and_scratch[0]
            rest = outs_and_scratch[1:]
        (src_v, dst_v, rbuf, zdbuf, ones_v, acc, dacc,
         sem_g0, sem_g1, sem_s, sem_s2, sem_d) = rest

        c = lax.axis_index("c")
        t = lax.axis_index("s")

        @pl.when(c == 0)
        def _features():
            for r in range(R):
                plsc.subcore_barrier()
                def zrow(i, _):
                    for jj in range(D // 16):
                        rbuf[0, i, pl.ds(jj * 16, 16)] = jnp.zeros((16,), _f32)
                    return 0
                lax.fori_loop(0, CHUNK, zrow, 0)

                def zero_it(k, _):
                    pltpu.sync_copy(
                        rbuf.at[0], acc.at[pl.ds(t * ROWS_PT + k * CHUNK, CHUNK)]
                    )
                    return 0
                lax.fori_loop(0, ROWS_PT // CHUNK, zero_it, 0)
                pltpu.sync_copy(src_h.at[r, t], src_v)
                pltpu.sync_copy(dst_h.at[r, t], dst_v)
                plsc.subcore_barrier()

                pltpu.async_copy(feat_h.at[src_v.at[0]], rbuf.at[0], sem_g0)

                def pair(k, _):
                    j0 = 2 * k
                    j1 = j0 + 1
                    pltpu.make_async_copy(
                        feat_h.at[src_v.at[j0]], rbuf.at[0], sem_g0
                    ).wait()
                    d_g1 = pltpu.async_copy(
                        feat_h.at[src_v.at[j1]], rbuf.at[1], sem_g1
                    )
                    d_s0 = pltpu.async_copy(
                        rbuf.at[0], acc.at[dst_v.at[j0]], sem_s, add=True
                    )
                    d_g1.wait()
                    d_s1 = pltpu.async_copy(
                        rbuf.at[1], acc.at[dst_v.at[j1]], sem_s2, add=True
                    )
                    d_s0.wait()
                    pltpu.async_copy(
                        feat_h.at[src_v.at[j0 + 2]], rbuf.at[0], sem_g0
                    )
                    d_s1.wait()
                    return 0
                lax.fori_loop(0, (CH - 1) // 2, pair, 0)
                pltpu.make_async_copy(
                    feat_h.at[src_v.at[CH - 1]], rbuf.at[0], sem_g0
                ).wait()
                pltpu.sync_copy(rbuf.at[0], acc.at[dst_v.at[CH - 1]], add=True)
                plsc.subcore_barrier()
                pltpu.sync_copy(
                    acc.at[pl.ds(t * ROWS_PT, ROWS_PT)],
                    out_agg.at[r, pl.ds(t * ROWS_PT, ROWS_PT)],
                )

        if with_deg:
            @pl.when(c == 1)
            def _degree():
                def fill1d(i, _):
                    zdbuf[pl.ds(i * 16, 16)] = jnp.zeros((16,), _f32)
                    return 0
                lax.fori_loop(0, ROWS_PT // 16, fill1d, 0)
                for j in range(CHUNK // 16):
                    ones_v[pl.ds(j * 16, 16)] = jnp.ones((16,), _f32)

                for r in range(R):
                    plsc.subcore_barrier()
                    pltpu.sync_copy(zdbuf, dacc.at[pl.ds(t * ROWS_PT, ROWS_PT)])
                    pltpu.sync_copy(dst_h.at[r, t], dst_v)
                    plsc.subcore_barrier()

                    def group(g, _):
                        descs = [
                            pltpu.async_copy(
                                ones_v, dacc.at[dst_v.at[g * 7 + u]],
                                sem_d, add=True,
                            )
                            for u in range(7)
                        ]
                        for dsc in descs:
                            dsc.wait()
                        return 0
                    lax.fori_loop(0, CH // 7, group, 0)
                    plsc.subcore_barrier()
                    pltpu.sync_copy(
                        dacc.at[pl.ds(t * ROWS_PT, ROWS_PT)],
                        out_deg.at[r, 0, pl.ds(t * ROWS_PT, ROWS_PT)],
                    )

    return body


def _sc_gather_mul(srcp, dstp, feat):
    out_type = jax.ShapeDtypeStruct((EPAD, D), _f32)
    scratch = [
        pltpu.VMEM((CH, CHUNK), jnp.int32),
        pltpu.VMEM((CH, CHUNK), jnp.int32),
        pltpu.VMEM((2, CHUNK, D), _f32),
        pltpu.VMEM((2, CHUNK, D), _f32),
        pltpu.VMEM((2, CHUNK, D), _f32),
        pltpu.SemaphoreType.DMA,
        pltpu.SemaphoreType.DMA,
        pltpu.SemaphoreType.DMA,
    ]

    @functools.partial(
        pl.kernel, out_type=out_type, mesh=_mesh(), scratch_types=scratch
    )
    def body(src_h, dst_h, feat_h, out_c, src_v, dst_v, bufa, bufb, bufc,
             sem_ga, sem_gb, sem_oc):
        c = lax.axis_index("c")
        s = lax.axis_index("s")
        base = s * CH * CHUNK

        @pl.when(c == 0)
        def _gathers():
            pltpu.sync_copy(src_h.at[s], src_v)
            pltpu.sync_copy(dst_h.at[s], dst_v)

            pltpu.async_copy(feat_h.at[src_v.at[0]], bufa.at[0], sem_ga)
            pltpu.async_copy(feat_h.at[dst_v.at[0]], bufb.at[0], sem_gb)

            def step(j, _):
                b = j & 1
                nb = 1 - b
                pltpu.make_async_copy(
                    feat_h.at[src_v.at[j]], bufa.at[b], sem_ga
                ).wait()
                pltpu.make_async_copy(
                    feat_h.at[dst_v.at[j]], bufb.at[b], sem_gb
                ).wait()

                @pl.when(j >= 1)
                def _():
                    pltpu.make_async_copy(
                        bufc.at[nb],
                        out_c.at[pl.ds(base + (j - 1) * CHUNK, CHUNK)], sem_oc,
                    ).wait()

                @pl.when(j + 1 < CH)
                def _():
                    pltpu.async_copy(
                        feat_h.at[src_v.at[j + 1]], bufa.at[nb], sem_ga
                    )
                    pltpu.async_copy(
                        feat_h.at[dst_v.at[j + 1]], bufb.at[nb], sem_gb
                    )

                def prow(i, _):
                    for kk in range(D // 16):
                        sl = pl.ds(kk * 16, 16)
                        bufc[b, i, sl] = bufa[b, i, sl] * bufb[b, i, sl]
                    return 0
                lax.fori_loop(0, CHUNK, prow, 0)

                pltpu.async_copy(
                    bufc.at[b], out_c.at[pl.ds(base + j * CHUNK, CHUNK)], sem_oc
                )
                return 0
            lax.fori_loop(0, CH, step, 0)
            lastb = (CH - 1) & 1
            pltpu.make_async_copy(
                bufc.at[lastb],
                out_c.at[pl.ds(base + (CH - 1) * CHUNK, CHUNK)], sem_oc,
            ).wait()

    return body(srcp, dstp, feat)


def _tc_layer(agg, deg2d, Ws, bs, relu):
    R = Ws.shape[0]
    BR = 1024
    DB = BR // D
    grid = (NP // BR,)

    def body(agg_ref, deg_ref, w_ref, b_ref, out_ref):
        rowsel = (
            lax.broadcasted_iota(jnp.int32, (BR, DB), 0) // D
            == lax.broadcasted_iota(jnp.int32, (BR, DB), 1)
        ).astype(_f32)
        lanesel = (
            lax.broadcasted_iota(jnp.int32, (BR, D), 0) % D
            == lax.broadcasted_iota(jnp.int32, (BR, D), 1)
        ).astype(_f32)
        acc = jnp.zeros((BR, D), _f32)
        for r in range(R):
            inv = 1.0 / jnp.maximum(deg_ref[r], 1.0)
            invrow = jnp.dot(rowsel, inv, preferred_element_type=_f32)
            invcol = jnp.sum(invrow * lanesel, axis=1, keepdims=True)
            a = agg_ref[r] * invcol
            m = jnp.dot(a, w_ref[r], preferred_element_type=_f32)
            m = m + b_ref[r][None, :]
            if relu:
                m = jnp.maximum(m, 0.0)
            acc = acc + m
        out_ref[...] = acc

    return pl.pallas_call(
        body,
        grid=grid,
        in_specs=[
            pl.BlockSpec((R, BR, D), lambda i: (0, i, 0)),
            pl.BlockSpec((R, DB, D), lambda i: (0, i, 0)),
            pl.BlockSpec((R, D, D), lambda i: (0, 0, 0)),
            pl.BlockSpec((R, D), lambda i: (0, 0)),
        ],
        out_specs=pl.BlockSpec((BR, D), lambda i: (i, 0)),
        out_shape=jax.ShapeDtypeStruct((NP, D), _f32),
    )(agg, deg2d, Ws, bs)


def _tc_predict(hp, P1, p1b, dvec, dbias):
    BR = 2048
    NBLK = EPAD // BR

    def body(hp_ref, p1_ref, p1b_ref, dv_ref, c_ref, out_ref):
        z = jnp.dot(hp_ref[...], p1_ref[...], preferred_element_type=_f32)
        z = jnp.maximum(z + p1b_ref[...], 0.0)
        logit = jnp.sum(z * dv_ref[...], axis=1) + c_ref[0]
        out_ref[...] = jax.nn.sigmoid(logit).reshape(BR // 256, 256)

    out = pl.pallas_call(
        body,
        grid=(NBLK,),
        in_specs=[
            pl.BlockSpec((BR, D), lambda i: (i, 0)),
            pl.BlockSpec((D, D), lambda i: (0, 0)),
            pl.BlockSpec((1, D), lambda i: (0, 0)),
            pl.BlockSpec((1, D), lambda i: (0, 0)),
            pl.BlockSpec(memory_space=pltpu.SMEM),
        ],
        out_specs=pl.BlockSpec((BR // 256, 256), lambda i: (i, 0)),
        out_shape=jax.ShapeDtypeStruct((EPAD // 256, 256), _f32),
    )(hp, P1, p1b, dvec, dbias)
    return out.reshape(EPAD)


def _pack_one(v, fill):
    pad = EPAD - v.shape[0]
    vp = jnp.concatenate([v, jnp.full((pad,), fill, jnp.int32)])
    return vp.reshape(NS, CH, CHUNK)


def _pack_edges(src, dst):
    return (_pack_one(src, 0), _pack_one(dst, N))


E_POS_CNT = 50000
E_NEG_CNT = 50000


def kernel(x, edge_r0, edge_r1, edge_r2, pos_edge, neg_edge,
           W1_0, b1_0, W1_1, b1_1, W1_2, b1_2,
           W2_0, b2_0, W2_1, b2_1, W2_2, b2_2,
           P1, p1b, P2, p2b):
    edges = [edge_r0, edge_r1, edge_r2]
    packed = [_pack_edges(e[0], e[1]) for e in edges]
    src3 = jnp.stack([p[0] for p in packed])
    dst3 = jnp.stack([p[1] for p in packed])

    W1s = jnp.stack([W1_0, W1_1, W1_2])
    b1s = jnp.stack([b1_0, b1_1, b1_2])
    W2s = jnp.stack([W2_0, W2_1, W2_2])
    b2s = jnp.stack([b2_0, b2_1, b2_2])

    agg1, deg = _sc_aggregate(src3, dst3, x, with_deg=True)(src3, dst3, x)
    deg2d = deg.reshape(3, NP // D, D)
    h1 = _tc_layer(agg1, deg2d, W1s, b1s, relu=True)
    res2 = _sc_aggregate(src3, dst3, h1, with_deg=False)(src3, dst3, h1)
    agg2 = res2[0] if isinstance(res2, (tuple, list)) else res2
    h2 = _tc_layer(agg2, deg2d, W2s, b2s, relu=False)

    src_sc = jnp.concatenate([pos_edge[0], neg_edge[0]])
    dst_sc = jnp.concatenate([pos_edge[1], neg_edge[1]])
    srcp, dstp = _pack_edges(src_sc, dst_sc)
    dstp = jnp.where(dstp >= N, 0, dstp)

    hp = _sc_gather_mul(srcp, dstp, h2)

    dvec = (P2[:, 1] - P2[:, 0]).reshape(1, D)
    dbias = (p2b[1] - p2b[0]).reshape(1)
    scores = _tc_predict(hp, P1, p1b.reshape(1, D), dvec, dbias)
    return (scores[:E_POS_CNT], scores[E_POS_CNT:E_POS_CNT + E_NEG_CNT])


# speedup vs baseline: 1.3768x; 1.3768x over previous
import functools

import jax
import jax.numpy as jnp
from jax import lax
from jax.experimental import pallas as pl
from jax.experimental.pallas import tpu as pltpu
from jax.experimental.pallas import tpu_sc as plsc

N = 10000
NP = 10240
D = 128
NC = 2
NS = 16
CHUNK = 128
CH = 49
EPAD = NS * CH * CHUNK
ROWS_PT = NP // NS

_f32 = jnp.float32


def _mesh():
    return plsc.VectorSubcoreMesh(
        core_axis_name="c", subcore_axis_name="s", num_cores=NC, num_subcores=NS
    )


def _sc_aggregate(src3, dst3, feat, with_deg):
    R = src3.shape[0]
    out_type = [jax.ShapeDtypeStruct((R, NP, D), _f32)]
    if with_deg:
        out_type.append(jax.ShapeDtypeStruct((R, 1, NP), _f32))

    scratch = [
        pltpu.VMEM((CH, CHUNK), jnp.int32),
        pltpu.VMEM((CH, CHUNK), jnp.int32),
        pltpu.VMEM((2, CHUNK, D), _f32),
        pltpu.VMEM((ROWS_PT,), _f32),
        pltpu.VMEM((CHUNK,), _f32),
        pltpu.VMEM_SHARED((NP, D), _f32),
        pltpu.VMEM_SHARED((NP,), _f32),
        pltpu.SemaphoreType.DMA,
        pltpu.SemaphoreType.DMA,
        pltpu.SemaphoreType.DMA,
        pltpu.SemaphoreType.DMA,
        pltpu.SemaphoreType.DMA,
    ]

    @functools.partial(
        pl.kernel, out_type=tuple(out_type), mesh=_mesh(), scratch_types=scratch
    )
    def body(src_h, dst_h, feat_h, *outs_and_scratch):
        if with_deg:
            out_agg, out_deg = outs_and_scratch[:2]
            rest = outs_and_scratch[2:]
        else:
            out_agg = outs_and_scratch[0]
            rest = outs_and_scratch[1:]
        (src_v, dst_v, rbuf, zdbuf, ones_v, acc, dacc,
         sem_g0, sem_g1, sem_s, sem_s2, sem_d) = rest

        c = lax.axis_index("c")
        t = lax.axis_index("s")

        @pl.when(c == 0)
        def _features():
            for r in range(R):
                plsc.subcore_barrier()
                def zrow(i, _):
                    for jj in range(D // 16):
                        rbuf[0, i, pl.ds(jj * 16, 16)] = jnp.zeros((16,), _f32)
                    return 0
                lax.fori_loop(0, CHUNK, zrow, 0)

                def zero_it(k, _):
                    pltpu.sync_copy(
                        rbuf.at[0], acc.at[pl.ds(t * ROWS_PT + k * CHUNK, CHUNK)]
                    )
                    return 0
                lax.fori_loop(0, ROWS_PT // CHUNK, zero_it, 0)
                pltpu.sync_copy(src_h.at[r, t], src_v)
                pltpu.sync_copy(dst_h.at[r, t], dst_v)
                plsc.subcore_barrier()

                pltpu.async_copy(feat_h.at[src_v.at[0]], rbuf.at[0], sem_g0)

                def pair(k, _):
                    j0 = 2 * k
                    j1 = j0 + 1
                    pltpu.make_async_copy(
                        feat_h.at[src_v.at[j0]], rbuf.at[0], sem_g0
                    ).wait()
                    d_g1 = pltpu.async_copy(
                        feat_h.at[src_v.at[j1]], rbuf.at[1], sem_g1
                    )
                    d_s0 = pltpu.async_copy(
                        rbuf.at[0], acc.at[dst_v.at[j0]], sem_s, add=True
                    )
                    d_s0.wait()
                    pltpu.async_copy(
                        feat_h.at[src_v.at[j0 + 2]], rbuf.at[0], sem_g0
                    )
                    d_g1.wait()
                    d_s1 = pltpu.async_copy(
                        rbuf.at[1], acc.at[dst_v.at[j1]], sem_s2, add=True
                    )
                    d_s1.wait()
                    return 0
                lax.fori_loop(0, (CH - 1) // 2, pair, 0)
                pltpu.make_async_copy(
                    feat_h.at[src_v.at[CH - 1]], rbuf.at[0], sem_g0
                ).wait()
                pltpu.sync_copy(rbuf.at[0], acc.at[dst_v.at[CH - 1]], add=True)
                plsc.subcore_barrier()
                pltpu.sync_copy(
                    acc.at[pl.ds(t * ROWS_PT, ROWS_PT)],
                    out_agg.at[r, pl.ds(t * ROWS_PT, ROWS_PT)],
                )

        if with_deg:
            @pl.when(c == 1)
            def _degree():
                def fill1d(i, _):
                    zdbuf[pl.ds(i * 16, 16)] = jnp.zeros((16,), _f32)
                    return 0
                lax.fori_loop(0, ROWS_PT // 16, fill1d, 0)
                for j in range(CHUNK // 16):
                    ones_v[pl.ds(j * 16, 16)] = jnp.ones((16,), _f32)

                for r in range(R):
                    plsc.subcore_barrier()
                    pltpu.sync_copy(zdbuf, dacc.at[pl.ds(t * ROWS_PT, ROWS_PT)])
                    pltpu.sync_copy(dst_h.at[r, t], dst_v)
                    plsc.subcore_barrier()

                    def group(g, _):
                        descs = [
                            pltpu.async_copy(
                                ones_v, dacc.at[dst_v.at[g * 7 + u]],
                                sem_d, add=True,
                            )
                            for u in range(7)
                        ]
                        for dsc in descs:
                            dsc.wait()
                        return 0
                    lax.fori_loop(0, CH // 7, group, 0)
                    plsc.subcore_barrier()
                    pltpu.sync_copy(
                        dacc.at[pl.ds(t * ROWS_PT, ROWS_PT)],
                        out_deg.at[r, 0, pl.ds(t * ROWS_PT, ROWS_PT)],
                    )

    return body


def _sc_gather_mul(srcp, dstp, feat):
    out_type = jax.ShapeDtypeStruct((EPAD, D), _f32)
    scratch = [
        pltpu.VMEM((CH, CHUNK), jnp.int32),
        pltpu.VMEM((CH, CHUNK), jnp.int32),
        pltpu.VMEM((2, CHUNK, D), _f32),
        pltpu.VMEM((2, CHUNK, D), _f32),
        pltpu.VMEM((2, CHUNK, D), _f32),
        pltpu.SemaphoreType.DMA,
        pltpu.SemaphoreType.DMA,
        pltpu.SemaphoreType.DMA,
    ]

    @functools.partial(
        pl.kernel, out_type=out_type, mesh=_mesh(), scratch_types=scratch
    )
    def body(src_h, dst_h, feat_h, out_c, src_v, dst_v, bufa, bufb, bufc,
             sem_ga, sem_gb, sem_oc):
        c = lax.axis_index("c")
        s = lax.axis_index("s")
        base = s * CH * CHUNK

        @pl.when(c == 0)
        def _gathers():
            pltpu.sync_copy(src_h.at[s], src_v)
            pltpu.sync_copy(dst_h.at[s], dst_v)

            pltpu.async_copy(feat_h.at[src_v.at[0]], bufa.at[0], sem_ga)
            pltpu.async_copy(feat_h.at[dst_v.at[0]], bufb.at[0], sem_gb)

            def step(j, _):
                b = j & 1
                nb = 1 - b
                pltpu.make_async_copy(
                    feat_h.at[src_v.at[j]], bufa.at[b], sem_ga
                ).wait()
                pltpu.make_async_copy(
                    feat_h.at[dst_v.at[j]], bufb.at[b], sem_gb
                ).wait()

                @pl.when(j >= 1)
                def _():
                    pltpu.make_async_copy(
                        bufc.at[nb],
                        out_c.at[pl.ds(base + (j - 1) * CHUNK, CHUNK)], sem_oc,
                    ).wait()

                @pl.when(j + 1 < CH)
                def _():
                    pltpu.async_copy(
                        feat_h.at[src_v.at[j + 1]], bufa.at[nb], sem_ga
                    )
                    pltpu.async_copy(
                        feat_h.at[dst_v.at[j + 1]], bufb.at[nb], sem_gb
                    )

                @plsc.parallel_loop(0, CHUNK, step=1, unroll=4)
                def _prow(i):
                    for kk in range(D // 16):
                        sl = pl.ds(kk * 16, 16)
                        bufc[b, i, sl] = bufa[b, i, sl] * bufb[b, i, sl]

                pltpu.async_copy(
                    bufc.at[b], out_c.at[pl.ds(base + j * CHUNK, CHUNK)], sem_oc
                )
                return 0
            lax.fori_loop(0, CH, step, 0)
            lastb = (CH - 1) & 1
            pltpu.make_async_copy(
                bufc.at[lastb],
                out_c.at[pl.ds(base + (CH - 1) * CHUNK, CHUNK)], sem_oc,
            ).wait()

    return body(srcp, dstp, feat)


def _tc_layer(agg, deg2d, Ws, bs, relu):
    R = Ws.shape[0]
    BR = 1024
    DB = BR // D
    grid = (NP // BR,)

    def body(agg_ref, deg_ref, w_ref, b_ref, out_ref):
        rowsel = (
            lax.broadcasted_iota(jnp.int32, (BR, DB), 0) // D
            == lax.broadcasted_iota(jnp.int32, (BR, DB), 1)
        ).astype(_f32)
        lanesel = (
            lax.broadcasted_iota(jnp.int32, (BR, D), 0) % D
            == lax.broadcasted_iota(jnp.int32, (BR, D), 1)
        ).astype(_f32)
        acc = jnp.zeros((BR, D), _f32)
        for r in range(R):
            inv = 1.0 / jnp.maximum(deg_ref[r], 1.0)
            invrow = jnp.dot(rowsel, inv, preferred_element_type=_f32)
            invcol = jnp.sum(invrow * lanesel, axis=1, keepdims=True)
            a = agg_ref[r] * invcol
            m = jnp.dot(a, w_ref[r], preferred_element_type=_f32)
            m = m + b_ref[r][None, :]
            if relu:
                m = jnp.maximum(m, 0.0)
            acc = acc + m
        out_ref[...] = acc

    return pl.pallas_call(
        body,
        grid=grid,
        in_specs=[
            pl.BlockSpec((R, BR, D), lambda i: (0, i, 0)),
            pl.BlockSpec((R, DB, D), lambda i: (0, i, 0)),
            pl.BlockSpec((R, D, D), lambda i: (0, 0, 0)),
            pl.BlockSpec((R, D), lambda i: (0, 0)),
        ],
        out_specs=pl.BlockSpec((BR, D), lambda i: (i, 0)),
        out_shape=jax.ShapeDtypeStruct((NP, D), _f32),
    )(agg, deg2d, Ws, bs)


def _tc_predict(hp, P1, p1b, dvec, dbias):
    BR = 2048
    NBLK = EPAD // BR

    def body(hp_ref, p1_ref, p1b_ref, dv_ref, c_ref, out_ref):
        z = jnp.dot(hp_ref[...], p1_ref[...], preferred_element_type=_f32)
        z = jnp.maximum(z + p1b_ref[...], 0.0)
        logit = jnp.sum(z * dv_ref[...], axis=1) + c_ref[0]
        out_ref[...] = jax.nn.sigmoid(logit).reshape(BR // 256, 256)

    out = pl.pallas_call(
        body,
        grid=(NBLK,),
        in_specs=[
            pl.BlockSpec((BR, D), lambda i: (i, 0)),
            pl.BlockSpec((D, D), lambda i: (0, 0)),
            pl.BlockSpec((1, D), lambda i: (0, 0)),
            pl.BlockSpec((1, D), lambda i: (0, 0)),
            pl.BlockSpec(memory_space=pltpu.SMEM),
        ],
        out_specs=pl.BlockSpec((BR // 256, 256), lambda i: (i, 0)),
        out_shape=jax.ShapeDtypeStruct((EPAD // 256, 256), _f32),
    )(hp, P1, p1b, dvec, dbias)
    return out.reshape(EPAD)


def _pack_one(v, fill):
    pad = EPAD - v.shape[0]
    vp = jnp.concatenate([v, jnp.full((pad,), fill, jnp.int32)])
    return vp.reshape(NS, CH, CHUNK)


def _pack_edges(src, dst):
    return (_pack_one(src, 0), _pack_one(dst, N))


E_POS_CNT = 50000
E_NEG_CNT = 50000


def kernel(x, edge_r0, edge_r1, edge_r2, pos_edge, neg_edge,
           W1_0, b1_0, W1_1, b1_1, W1_2, b1_2,
           W2_0, b2_0, W2_1, b2_1, W2_2, b2_2,
           P1, p1b, P2, p2b):
    edges = [edge_r0, edge_r1, edge_r2]
    packed = [_pack_edges(e[0], e[1]) for e in edges]
    src3 = jnp.stack([p[0] for p in packed])
    dst3 = jnp.stack([p[1] for p in packed])

    W1s = jnp.stack([W1_0, W1_1, W1_2])
    b1s = jnp.stack([b1_0, b1_1, b1_2])
    W2s = jnp.stack([W2_0, W2_1, W2_2])
    b2s = jnp.stack([b2_0, b2_1, b2_2])

    agg1, deg = _sc_aggregate(src3, dst3, x, with_deg=True)(src3, dst3, x)
    deg2d = deg.reshape(3, NP // D, D)
    h1 = _tc_layer(agg1, deg2d, W1s, b1s, relu=True)
    res2 = _sc_aggregate(src3, dst3, h1, with_deg=False)(src3, dst3, h1)
    agg2 = res2[0] if isinstance(res2, (tuple, list)) else res2
    h2 = _tc_layer(agg2, deg2d, W2s, b2s, relu=False)

    src_sc = jnp.concatenate([pos_edge[0], neg_edge[0]])
    dst_sc = jnp.concatenate([pos_edge[1], neg_edge[1]])
    srcp, dstp = _pack_edges(src_sc, dst_sc)
    dstp = jnp.where(dstp >= N, 0, dstp)

    hp = _sc_gather_mul(srcp, dstp, h2)

    dvec = (P2[:, 1] - P2[:, 0]).reshape(1, D)
    dbias = (p2b[1] - p2b[0]).reshape(1)
    scores = _tc_predict(hp, P1, p1b.reshape(1, D), dvec, dbias)
    return (scores[:E_POS_CNT], scores[E_POS_CNT:E_POS_CNT + E_NEG_CNT])


# speedup vs baseline: 1.4304x; 1.0389x over previous
import functools

import jax
import jax.numpy as jnp
from jax import lax
from jax.experimental import pallas as pl
from jax.experimental.pallas import tpu as pltpu
from jax.experimental.pallas import tpu_sc as plsc

N = 10000
NP = 10240
D = 128
NC = 2
NS = 16
CHUNK = 128
CH = 49
EPAD = NS * CH * CHUNK
ROWS_PT = NP // NS

_f32 = jnp.float32


def _mesh():
    return plsc.VectorSubcoreMesh(
        core_axis_name="c", subcore_axis_name="s", num_cores=NC, num_subcores=NS
    )


def _sc_aggregate(src3, dst3, feat, with_deg):
    R = src3.shape[0]
    out_type = [jax.ShapeDtypeStruct((R, NP, D), _f32)]
    if with_deg:
        out_type.append(jax.ShapeDtypeStruct((R, 1, NP), _f32))

    scratch = [
        pltpu.VMEM((CH, CHUNK), jnp.int32),
        pltpu.VMEM((CH, CHUNK), jnp.int32),
        pltpu.VMEM((2, CHUNK, D), _f32),
        pltpu.VMEM((ROWS_PT,), _f32),
        pltpu.VMEM((CHUNK,), _f32),
        pltpu.VMEM_SHARED((NP, D), _f32),
        pltpu.VMEM_SHARED((NP,), _f32),
        pltpu.SemaphoreType.DMA,
        pltpu.SemaphoreType.DMA,
        pltpu.SemaphoreType.DMA,
        pltpu.SemaphoreType.DMA,
        pltpu.SemaphoreType.DMA,
    ]

    @functools.partial(
        pl.kernel, out_type=tuple(out_type), mesh=_mesh(), scratch_types=scratch
    )
    def body(src_h, dst_h, feat_h, *outs_and_scratch):
        if with_deg:
            out_agg, out_deg = outs_and_scratch[:2]
            rest = outs_and_scratch[2:]
        else:
            out_agg = outs_and_scratch[0]
            rest = outs_and_scratch[1:]
        (src_v, dst_v, rbuf, zdbuf, ones_v, acc, dacc,
         sem_g0, sem_g1, sem_s, sem_s2, sem_d) = rest

        c = lax.axis_index("c")
        t = lax.axis_index("s")

        @pl.when(c == 0)
        def _features():
            for r in range(R):
                plsc.subcore_barrier()
                def zrow(i, _):
                    for jj in range(D // 16):
                        rbuf[0, i, pl.ds(jj * 16, 16)] = jnp.zeros((16,), _f32)
                    return 0
                lax.fori_loop(0, CHUNK, zrow, 0)

                def zero_it(k, _):
                    pltpu.sync_copy(
                        rbuf.at[0], acc.at[pl.ds(t * ROWS_PT + k * CHUNK, CHUNK)]
                    )
                    return 0
                lax.fori_loop(0, ROWS_PT // CHUNK, zero_it, 0)
                pltpu.sync_copy(src_h.at[r, t], src_v)
                pltpu.sync_copy(dst_h.at[r, t], dst_v)
                plsc.subcore_barrier()

                pltpu.async_copy(feat_h.at[src_v.at[0]], rbuf.at[0], sem_g0)

                def pair(k, _):
                    j0 = 2 * k
                    j1 = j0 + 1
                    pltpu.make_async_copy(
                        feat_h.at[src_v.at[j0]], rbuf.at[0], sem_g0
                    ).wait()
                    d_g1 = pltpu.async_copy(
                        feat_h.at[src_v.at[j1]], rbuf.at[1], sem_g1
                    )
                    d_s0 = pltpu.async_copy(
                        rbuf.at[0], acc.at[dst_v.at[j0]], sem_s, add=True
                    )
                    d_s0.wait()
                    pltpu.async_copy(
                        feat_h.at[src_v.at[j0 + 2]], rbuf.at[0], sem_g0
                    )
                    d_g1.wait()
                    d_s1 = pltpu.async_copy(
                        rbuf.at[1], acc.at[dst_v.at[j1]], sem_s2, add=True
                    )
                    d_s1.wait()
                    return 0
                lax.fori_loop(0, (CH - 1) // 2, pair, 0)
                pltpu.make_async_copy(
                    feat_h.at[src_v.at[CH - 1]], rbuf.at[0], sem_g0
                ).wait()
                pltpu.sync_copy(rbuf.at[0], acc.at[dst_v.at[CH - 1]], add=True)
                plsc.subcore_barrier()
                pltpu.sync_copy(
                    acc.at[pl.ds(t * ROWS_PT, ROWS_PT)],
                    out_agg.at[r, pl.ds(t * ROWS_PT, ROWS_PT)],
                )

        if with_deg:
            @pl.when(c == 1)
            def _degree():
                def fill1d(i, _):
                    zdbuf[pl.ds(i * 16, 16)] = jnp.zeros((16,), _f32)
                    return 0
                lax.fori_loop(0, ROWS_PT // 16, fill1d, 0)
                for j in range(CHUNK // 16):
                    ones_v[pl.ds(j * 16, 16)] = jnp.ones((16,), _f32)

                for r in range(R):
                    plsc.subcore_barrier()
                    pltpu.sync_copy(zdbuf, dacc.at[pl.ds(t * ROWS_PT, ROWS_PT)])
                    pltpu.sync_copy(dst_h.at[r, t], dst_v)
                    plsc.subcore_barrier()

                    def group(g, _):
                        descs = [
                            pltpu.async_copy(
                                ones_v, dacc.at[dst_v.at[g * 7 + u]],
                                sem_d, add=True,
                            )
                            for u in range(7)
                        ]
                        for dsc in descs:
                            dsc.wait()
                        return 0
                    lax.fori_loop(0, CH // 7, group, 0)
                    plsc.subcore_barrier()
                    pltpu.sync_copy(
                        dacc.at[pl.ds(t * ROWS_PT, ROWS_PT)],
                        out_deg.at[r, 0, pl.ds(t * ROWS_PT, ROWS_PT)],
                    )

    return body


def _sc_gather_mul(srcp, dstp, feat):
    out_type = jax.ShapeDtypeStruct((EPAD, D), _f32)
    scratch = [
        pltpu.VMEM((CH, CHUNK), jnp.int32),
        pltpu.VMEM((CH, CHUNK), jnp.int32),
        pltpu.VMEM((2, CHUNK, D), _f32),
        pltpu.VMEM((2, CHUNK, D), _f32),
        pltpu.VMEM((2, CHUNK, D), _f32),
        pltpu.SemaphoreType.DMA,
        pltpu.SemaphoreType.DMA,
        pltpu.SemaphoreType.DMA,
    ]

    @functools.partial(
        pl.kernel, out_type=out_type, mesh=_mesh(), scratch_types=scratch
    )
    def body(src_h, dst_h, feat_h, out_c, src_v, dst_v, bufa, bufb, bufc,
             sem_ga, sem_gb, sem_oc):
        c = lax.axis_index("c")
        s = lax.axis_index("s")
        base = s * CH * CHUNK

        @pl.when(c == 0)
        def _gathers():
            pltpu.sync_copy(src_h.at[s], src_v)
            pltpu.sync_copy(dst_h.at[s], dst_v)

            pltpu.async_copy(feat_h.at[src_v.at[0]], bufa.at[0], sem_ga)
            pltpu.async_copy(feat_h.at[dst_v.at[0]], bufb.at[0], sem_gb)

            def step(j, _):
                b = j & 1
                nb = 1 - b
                pltpu.make_async_copy(
                    feat_h.at[src_v.at[j]], bufa.at[b], sem_ga
                ).wait()
                pltpu.make_async_copy(
                    feat_h.at[dst_v.at[j]], bufb.at[b], sem_gb
                ).wait()

                @pl.when(j >= 1)
                def _():
                    pltpu.make_async_copy(
                        bufc.at[nb],
                        out_c.at[pl.ds(base + (j - 1) * CHUNK, CHUNK)], sem_oc,
                    ).wait()

                @pl.when(j + 1 < CH)
                def _():
                    pltpu.async_copy(
                        feat_h.at[src_v.at[j + 1]], bufa.at[nb], sem_ga
                    )
                    pltpu.async_copy(
                        feat_h.at[dst_v.at[j + 1]], bufb.at[nb], sem_gb
                    )

                @plsc.parallel_loop(0, CHUNK, step=1, unroll=4)
                def _prow(i):
                    for kk in range(D // 16):
                        sl = pl.ds(kk * 16, 16)
                        bufc[b, i, sl] = bufa[b, i, sl] * bufb[b, i, sl]

                pltpu.async_copy(
                    bufc.at[b], out_c.at[pl.ds(base + j * CHUNK, CHUNK)], sem_oc
                )
                return 0
            lax.fori_loop(0, CH, step, 0)
            lastb = (CH - 1) & 1
            pltpu.make_async_copy(
                bufc.at[lastb],
                out_c.at[pl.ds(base + (CH - 1) * CHUNK, CHUNK)], sem_oc,
            ).wait()

    return body(srcp, dstp, feat)


def _tc_layer(agg, deg2d, Ws, bs, relu):
    R = Ws.shape[0]
    BR = 2048
    DB = BR // D
    grid = (NP // BR,)

    def body(agg_ref, deg_ref, w_ref, b_ref, out_ref):
        rowsel = (
            lax.broadcasted_iota(jnp.int32, (BR, DB), 0) // D
            == lax.broadcasted_iota(jnp.int32, (BR, DB), 1)
        ).astype(_f32)
        lanesel = (
            lax.broadcasted_iota(jnp.int32, (BR, D), 0) % D
            == lax.broadcasted_iota(jnp.int32, (BR, D), 1)
        ).astype(_f32)
        acc = jnp.zeros((BR, D), _f32)
        for r in range(R):
            inv = 1.0 / jnp.maximum(deg_ref[r], 1.0)
            invrow = jnp.dot(rowsel, inv, preferred_element_type=_f32)
            invcol = jnp.sum(invrow * lanesel, axis=1, keepdims=True)
            a = agg_ref[r] * invcol
            m = jnp.dot(a, w_ref[r], preferred_element_type=_f32)
            m = m + b_ref[r][None, :]
            if relu:
                m = jnp.maximum(m, 0.0)
            acc = acc + m
        out_ref[...] = acc

    return pl.pallas_call(
        body,
        grid=grid,
        in_specs=[
            pl.BlockSpec((R, BR, D), lambda i: (0, i, 0)),
            pl.BlockSpec((R, DB, D), lambda i: (0, i, 0)),
            pl.BlockSpec((R, D, D), lambda i: (0, 0, 0)),
            pl.BlockSpec((R, D), lambda i: (0, 0)),
        ],
        out_specs=pl.BlockSpec((BR, D), lambda i: (i, 0)),
        out_shape=jax.ShapeDtypeStruct((NP, D), _f32),
    )(agg, deg2d, Ws, bs)


def _tc_predict(hp, P1, p1b, dvec, dbias):
    BR = 14336
    NBLK = EPAD // BR

    def body(hp_ref, p1_ref, p1b_ref, dv_ref, c_ref, out_ref):
        z = jnp.dot(hp_ref[...], p1_ref[...], preferred_element_type=_f32)
        z = jnp.maximum(z + p1b_ref[...], 0.0)
        logit = jnp.sum(z * dv_ref[...], axis=1) + c_ref[0]
        out_ref[...] = jax.nn.sigmoid(logit).reshape(BR // 256, 256)

    out = pl.pallas_call(
        body,
        grid=(NBLK,),
        in_specs=[
            pl.BlockSpec((BR, D), lambda i: (i, 0)),
            pl.BlockSpec((D, D), lambda i: (0, 0)),
            pl.BlockSpec((1, D), lambda i: (0, 0)),
            pl.BlockSpec((1, D), lambda i: (0, 0)),
            pl.BlockSpec(memory_space=pltpu.SMEM),
        ],
        out_specs=pl.BlockSpec((BR // 256, 256), lambda i: (i, 0)),
        out_shape=jax.ShapeDtypeStruct((EPAD // 256, 256), _f32),
    )(hp, P1, p1b, dvec, dbias)
    return out.reshape(EPAD)


def _pack_one(v, fill):
    pad = EPAD - v.shape[0]
    vp = jnp.concatenate([v, jnp.full((pad,), fill, jnp.int32)])
    return vp.reshape(NS, CH, CHUNK)


def _pack_edges(src, dst):
    return (_pack_one(src, 0), _pack_one(dst, N))


E_POS_CNT = 50000
E_NEG_CNT = 50000


def kernel(x, edge_r0, edge_r1, edge_r2, pos_edge, neg_edge,
           W1_0, b1_0, W1_1, b1_1, W1_2, b1_2,
           W2_0, b2_0, W2_1, b2_1, W2_2, b2_2,
           P1, p1b, P2, p2b):
    edges = [edge_r0, edge_r1, edge_r2]
    packed = [_pack_edges(e[0], e[1]) for e in edges]
    src3 = jnp.stack([p[0] for p in packed])
    dst3 = jnp.stack([p[1] for p in packed])

    W1s = jnp.stack([W1_0, W1_1, W1_2])
    b1s = jnp.stack([b1_0, b1_1, b1_2])
    W2s = jnp.stack([W2_0, W2_1, W2_2])
    b2s = jnp.stack([b2_0, b2_1, b2_2])

    agg1, deg = _sc_aggregate(src3, dst3, x, with_deg=True)(src3, dst3, x)
    deg2d = deg.reshape(3, NP // D, D)
    h1 = _tc_layer(agg1, deg2d, W1s, b1s, relu=True)
    res2 = _sc_aggregate(src3, dst3, h1, with_deg=False)(src3, dst3, h1)
    agg2 = res2[0] if isinstance(res2, (tuple, list)) else res2
    h2 = _tc_layer(agg2, deg2d, W2s, b2s, relu=False)

    src_sc = jnp.concatenate([pos_edge[0], neg_edge[0]])
    dst_sc = jnp.concatenate([pos_edge[1], neg_edge[1]])
    srcp, dstp = _pack_edges(src_sc, dst_sc)
    dstp = jnp.where(dstp >= N, 0, dstp)

    hp = _sc_gather_mul(srcp, dstp, h2)

    dvec = (P2[:, 1] - P2[:, 0]).reshape(1, D)
    dbias = (p2b[1] - p2b[0]).reshape(1)
    scores = _tc_predict(hp, P1, p1b.reshape(1, D), dvec, dbias)
    return (scores[:E_POS_CNT], scores[E_POS_CNT:E_POS_CNT + E_NEG_CNT])
